# Initial kernel scaffold; baseline (speedup 1.0000x reference)
import jax
import jax.numpy as jnp
from jax.experimental import pallas as pl


def kernel(indices, table, pos_weight):
    raise NotImplementedError("write your pallas kernel here")


# speedup vs baseline: 12.7622x; 12.7622x over previous
import functools

import jax
import jax.numpy as jnp
from jax import lax
from jax.experimental import pallas as pl
from jax.experimental.pallas import tpu as pltpu
from jax.experimental.pallas import tpu_sc as plsc

_NC = 2
_NS = 16
_LANES = 16


def _build(F, B, L, V, D):
    NW = _NC * _NS
    NB = B // NW
    CHUNK = NB * L
    assert CHUNK % 128 == 0
    NGATHER = CHUNK // 128

    mesh = plsc.VectorSubcoreMesh(
        core_axis_name="c", subcore_axis_name="s",
        num_cores=_NC, num_subcores=_NS)

    @functools.partial(
        pl.kernel,
        out_type=jax.ShapeDtypeStruct((B, F, D), jnp.float32),
        mesh=mesh,
        compiler_params=pltpu.CompilerParams(use_tc_tiling_on_sc=False),
        scratch_types=[
            pltpu.VMEM((CHUNK,), jnp.int32),
            pltpu.VMEM((CHUNK, D), jnp.float32),
            pltpu.VMEM((NB, D), jnp.float32),
            pltpu.VMEM((F, L, _LANES), jnp.float32),
            pltpu.SemaphoreType.DMA,
        ],
    )
    def run(idx_hbm, table_hbm, pwe_hbm, out_hbm, idx_v, rows_v, out_v, pw_v,
            sem):
        wid = lax.axis_index("s") * _NC + lax.axis_index("c")
        pltpu.sync_copy(pwe_hbm, pw_v)

        def f_body(f, _):
            base_idx = f * (B * L) + wid * CHUNK
            pltpu.sync_copy(idx_hbm.at[pl.ds(base_idx, CHUNK)], idx_v)
            cps = [
                pltpu.async_copy(table_hbm.at[idx_v.at[pl.ds(j * 128, 128)]],
                                 rows_v.at[pl.ds(j * 128, 128)], sem)
                for j in range(NGATHER)
            ]
            for c in cps:
                c.wait()
            wv = [pw_v[f, l, :] for l in range(L)]

            def bag(i, _):
                base = i * L
                acc0 = jnp.zeros((_LANES,), jnp.float32)
                acc1 = jnp.zeros((_LANES,), jnp.float32)
                for l in range(L):
                    acc0 = acc0 + wv[l] * rows_v[base + l, 0:16]
                    acc1 = acc1 + wv[l] * rows_v[base + l, 16:32]
                out_v[i, 0:16] = acc0
                out_v[i, 16:32] = acc1
                return 0

            lax.fori_loop(0, NB, bag, 0)
            pltpu.sync_copy(out_v, out_hbm.at[pl.ds(wid * NB, NB), f])
            return 0

        lax.fori_loop(0, F, f_body, 0)

    return run


def kernel(indices, table, pos_weight):
    F, B, L = indices.shape
    V, D = table.shape
    idx_flat = indices.astype(jnp.int32).reshape(F * B * L)
    pwe = jnp.broadcast_to(
        pos_weight.astype(jnp.float32)[:, :, None], (F, L, _LANES))
    run = _build(F, B, L, V, D)
    out = run(idx_flat, table, pwe)
    return out.reshape(B, F * D)


# speedup vs baseline: 13.3096x; 1.0429x over previous
import functools

import jax
import jax.numpy as jnp
from jax import lax
from jax.experimental import pallas as pl
from jax.experimental.pallas import tpu as pltpu
from jax.experimental.pallas import tpu_sc as plsc

_NC = 2
_NS = 16
_LANES = 16


def _build(F, B, L, V, D):
    NW = _NC * _NS
    NB = B // NW
    CHUNK = NB * L
    assert CHUNK % 128 == 0
    NGATHER = CHUNK // 128

    mesh = plsc.VectorSubcoreMesh(
        core_axis_name="c", subcore_axis_name="s",
        num_cores=_NC, num_subcores=_NS)

    @functools.partial(
        pl.kernel,
        out_type=jax.ShapeDtypeStruct((B, F * D), jnp.float32),
        mesh=mesh,
        compiler_params=pltpu.CompilerParams(use_tc_tiling_on_sc=False),
        scratch_types=[
            pltpu.VMEM((CHUNK,), jnp.int32),
            pltpu.VMEM((CHUNK, D), jnp.float32),
            pltpu.VMEM((NB, D), jnp.float32),
            pltpu.VMEM((F, L, _LANES), jnp.float32),
            pltpu.SemaphoreType.DMA,
        ],
    )
    def run(idx_hbm, table_hbm, pwe_hbm, out_hbm, idx_v, rows_v, out_v, pw_v,
            sem):
        wid = lax.axis_index("s") * _NC + lax.axis_index("c")
        pltpu.sync_copy(pwe_hbm, pw_v)

        def f_body(f, _):
            base_idx = f * (B * L) + wid * CHUNK
            pltpu.sync_copy(idx_hbm.at[pl.ds(base_idx, CHUNK)], idx_v)
            cps = [
                pltpu.async_copy(table_hbm.at[idx_v.at[pl.ds(j * 128, 128)]],
                                 rows_v.at[pl.ds(j * 128, 128)], sem)
                for j in range(NGATHER)
            ]
            for c in cps:
                c.wait()
            wv = [pw_v[f, l, :] for l in range(L)]

            def bag(i, _):
                base = i * L
                acc0 = jnp.zeros((_LANES,), jnp.float32)
                acc1 = jnp.zeros((_LANES,), jnp.float32)
                for l in range(L):
                    acc0 = acc0 + wv[l] * rows_v[base + l, 0:16]
                    acc1 = acc1 + wv[l] * rows_v[base + l, 16:32]
                out_v[i, 0:16] = acc0
                out_v[i, 16:32] = acc1
                return 0

            lax.fori_loop(0, NB, bag, 0)
            pltpu.sync_copy(out_v, out_hbm.at[pl.ds(wid * NB, NB),
                                              pl.ds(f * D, D)])
            return 0

        lax.fori_loop(0, F, f_body, 0)

    return run


def kernel(indices, table, pos_weight):
    F, B, L = indices.shape
    V, D = table.shape
    idx_flat = indices.astype(jnp.int32).reshape(F * B * L)
    pwe = jnp.broadcast_to(
        pos_weight.astype(jnp.float32)[:, :, None], (F, L, _LANES))
    run = _build(F, B, L, V, D)
    return run(idx_flat, table, pwe)
